# speedup vs baseline: 1.5222x; 1.5222x over previous
import jax
import jax.numpy as jnp
from jax.experimental import pallas as pl


def _copy_body(x_ref, o_ref):
    o_ref[...] = x_ref[...]


def _pallas_copy(x):
    return pl.pallas_call(
        _copy_body,
        out_shape=jax.ShapeDtypeStruct(x.shape, x.dtype),
    )(x)


def _gat_layer(x, src, dst, ae, W, a_src, a_dst, b, N):
    h = x @ W
    asrc = h @ a_src
    adst = h @ a_dst
    m = jax.nn.leaky_relu(jnp.max(asrc) + jnp.max(adst) + jnp.max(ae), 0.2)
    alpha = asrc[src] + adst[dst] + ae
    alpha = jax.nn.leaky_relu(alpha, negative_slope=0.2)
    ex = jnp.exp(alpha - m)
    denom = jax.ops.segment_sum(ex, dst, num_segments=N)
    acc = jax.ops.segment_sum(h[src] * ex[:, None], dst, num_segments=N)
    out = acc / (denom + 1e-16)[:, None] + b
    return out


def kernel(x, edge_index, edge_attr, batch,
           W1, att_src1, att_dst1, We1, att_edge1, b1,
           W2, att_src2, att_dst2, We2, att_edge2, b2,
           gamma, beta):
    N, D = x.shape
    G = 16
    src = edge_index[0]
    dst = edge_index[1]
    ae1 = edge_attr @ (We1 @ att_edge1)
    ae2 = edge_attr @ (We2 @ att_edge2)

    x = _pallas_copy(x)

    h = _gat_layer(x, src, dst, ae1, W1, att_src1, att_dst1, b1, N)
    h = jax.nn.silu(h)
    h = h + x
    x2 = h
    h = _gat_layer(x2, src, dst, ae2, W2, att_src2, att_dst2, b2, N)
    h = jax.nn.silu(h)
    h = h + x2

    mu = jnp.mean(h, axis=-1, keepdims=True)
    var = jnp.var(h, axis=-1, keepdims=True)
    h = gamma * (h - mu) / jnp.sqrt(var + 1e-5) + beta

    onehot = (batch[None, :] == jnp.arange(G, dtype=jnp.int32)[:, None]).astype(jnp.float32)
    sums = onehot @ h
    counts = jnp.sum(onehot, axis=1)
    return sums / jnp.maximum(counts, 1.0)[:, None]


# speedup vs baseline: 14.5078x; 9.5310x over previous
import functools

import jax
import jax.numpy as jnp
from jax import lax
from jax.experimental import pallas as pl
from jax.experimental.pallas import tpu as pltpu
from jax.experimental.pallas import tpu_sc as plsc

N = 10000
E = 320000
D = 128
DE = 16
G = 16

DX = 144
NC, NS, L = 2, 16, 16
NW = NC * NS
EPW = E // NW
B = 80
NCH = EPW // B
STRIPE = N // NS
BN = 1000
NB = N // BN


def _silu(x):
    return x / (1.0 + jnp.exp(-x))


def _pre_body(x_ref, w_ref, as_ref, ad_ref, hext_ref, asrc_ref, adst_ref):
    h = jnp.dot(x_ref[...], w_ref[...], preferred_element_type=jnp.float32)
    hext_ref[:, :D] = h
    ones_col = (lax.broadcasted_iota(jnp.int32, (BN, DX - D), 1) == 0)
    hext_ref[:, D:] = ones_col.astype(jnp.float32)
    asrc_ref[...] = jnp.dot(h, as_ref[...], preferred_element_type=jnp.float32)
    adst_ref[...] = jnp.dot(h, ad_ref[...], preferred_element_type=jnp.float32)


def _pre(x, W, a_src, a_dst):
    return pl.pallas_call(
        _pre_body,
        grid=(NB,),
        in_specs=[
            pl.BlockSpec((BN, D), lambda i: (i, 0)),
            pl.BlockSpec((D, D), lambda i: (0, 0)),
            pl.BlockSpec((D, 1), lambda i: (0, 0)),
            pl.BlockSpec((D, 1), lambda i: (0, 0)),
        ],
        out_specs=[
            pl.BlockSpec((BN, DX), lambda i: (i, 0)),
            pl.BlockSpec((BN, 1), lambda i: (i, 0)),
            pl.BlockSpec((BN, 1), lambda i: (i, 0)),
        ],
        out_shape=[
            jax.ShapeDtypeStruct((N, DX), jnp.float32),
            jax.ShapeDtypeStruct((N, 1), jnp.float32),
            jax.ShapeDtypeStruct((N, 1), jnp.float32),
        ],
    )(x, W, a_src, a_dst)


def _mid_body(acc_ref, x_ref, b_ref, w_ref, as_ref, ad_ref,
              x2_ref, hext_ref, asrc_ref, adst_ref):
    s = acc_ref[0] + acc_ref[1]
    o = s[:, :D] / (s[:, D:D + 1] + 1e-16) + b_ref[...]
    x2 = _silu(o) + x_ref[...]
    x2_ref[...] = x2
    h2 = jnp.dot(x2, w_ref[...], preferred_element_type=jnp.float32)
    hext_ref[:, :D] = h2
    ones_col = (lax.broadcasted_iota(jnp.int32, (BN, DX - D), 1) == 0)
    hext_ref[:, D:] = ones_col.astype(jnp.float32)
    asrc_ref[...] = jnp.dot(h2, as_ref[...], preferred_element_type=jnp.float32)
    adst_ref[...] = jnp.dot(h2, ad_ref[...], preferred_element_type=jnp.float32)


def _mid(acc, x, b, W, a_src, a_dst):
    return pl.pallas_call(
        _mid_body,
        grid=(NB,),
        in_specs=[
            pl.BlockSpec((2, BN, DX), lambda i: (0, i, 0)),
            pl.BlockSpec((BN, D), lambda i: (i, 0)),
            pl.BlockSpec((1, D), lambda i: (0, 0)),
            pl.BlockSpec((D, D), lambda i: (0, 0)),
            pl.BlockSpec((D, 1), lambda i: (0, 0)),
            pl.BlockSpec((D, 1), lambda i: (0, 0)),
        ],
        out_specs=[
            pl.BlockSpec((BN, D), lambda i: (i, 0)),
            pl.BlockSpec((BN, DX), lambda i: (i, 0)),
            pl.BlockSpec((BN, 1), lambda i: (i, 0)),
            pl.BlockSpec((BN, 1), lambda i: (i, 0)),
        ],
        out_shape=[
            jax.ShapeDtypeStruct((N, D), jnp.float32),
            jax.ShapeDtypeStruct((N, DX), jnp.float32),
            jax.ShapeDtypeStruct((N, 1), jnp.float32),
            jax.ShapeDtypeStruct((N, 1), jnp.float32),
        ],
    )(acc, x, b, W, a_src, a_dst)


def _final_body(acc_ref, x_ref, b_ref, g_ref, be_ref, batch_ref,
                out_ref, pool_ref, cnt_ref):
    i = pl.program_id(0)
    s = acc_ref[0] + acc_ref[1]
    o = s[:, :D] / (s[:, D:D + 1] + 1e-16) + b_ref[...]
    h = _silu(o) + x_ref[...]
    mu = jnp.mean(h, axis=1, keepdims=True)
    var = jnp.mean((h - mu) ** 2, axis=1, keepdims=True)
    hn = g_ref[...] * (h - mu) * lax.rsqrt(var + 1e-5) + be_ref[...]
    onehot = (batch_ref[0] == lax.broadcasted_iota(jnp.int32, (G, BN), 0))
    onehot = onehot.astype(jnp.float32)
    part = jnp.dot(onehot, hn, preferred_element_type=jnp.float32)
    cpart = jnp.sum(onehot, axis=1)[:, None]

    @pl.when(i == 0)
    def _():
        pool_ref[...] = jnp.zeros((G, D), jnp.float32)
        cnt_ref[...] = jnp.zeros((G, 1), jnp.float32)

    pool_ref[...] += part
    cnt_ref[...] += cpart

    @pl.when(i == NB - 1)
    def _():
        out_ref[...] = pool_ref[...] / jnp.maximum(cnt_ref[...], 1.0)


def _final(acc, x2, b, gamma, beta, batch_row):
    return pl.pallas_call(
        _final_body,
        grid=(NB,),
        in_specs=[
            pl.BlockSpec((2, BN, DX), lambda i: (0, i, 0)),
            pl.BlockSpec((BN, D), lambda i: (i, 0)),
            pl.BlockSpec((1, D), lambda i: (0, 0)),
            pl.BlockSpec((1, D), lambda i: (0, 0)),
            pl.BlockSpec((1, D), lambda i: (0, 0)),
            pl.BlockSpec((1, 1, BN), lambda i: (i, 0, 0)),
        ],
        out_specs=pl.BlockSpec((G, D), lambda i: (0, 0)),
        out_shape=jax.ShapeDtypeStruct((G, D), jnp.float32),
        scratch_shapes=[
            pltpu.VMEM((G, D), jnp.float32),
            pltpu.VMEM((G, 1), jnp.float32),
        ],
    )(acc, x2, b, gamma, beta, batch_row)


def _sc_edge_body(hext_hbm, src_hbm, dst_hbm, ea_hbm, asrc_hbm, adst_hbm,
                  ve_hbm, acc_hbm,
                  srcv, dstv, asrcv, adstv, eav, rowsv, exbuf, vev,
                  accsh, sem):
    cid = lax.axis_index("c")
    sid = lax.axis_index("s")
    wid = sid * NC + cid

    pltpu.sync_copy(asrc_hbm, asrcv)
    pltpu.sync_copy(adst_hbm, adstv)
    pltpu.sync_copy(ve_hbm, vev)

    zero16 = jnp.zeros((L,), jnp.float32)

    def _zrow(r, _):
        for cb in range(DX // L):
            rowsv[r, pl.ds(cb * L, L)] = zero16
        return 0

    lax.fori_loop(0, B, _zrow, 0)
    for k in range(STRIPE // B):
        pltpu.sync_copy(rowsv, accsh.at[pl.ds(sid * STRIPE + k * B, B)])
    rem = STRIPE % B
    if rem:
        pltpu.sync_copy(rowsv.at[pl.ds(0, rem)],
                        accsh.at[pl.ds(sid * STRIPE + (STRIPE // B) * B, rem)])
    plsc.subcore_barrier()

    neg = jnp.full((L,), -3.0e38, jnp.float32)

    def _mred(i, carry):
        a, bmax = carry
        return (jnp.maximum(a, asrcv[pl.ds(i * L, L)]),
                jnp.maximum(bmax, adstv[pl.ds(i * L, L)]))

    am, bm = lax.fori_loop(0, N // L, _mred, (neg, neg))
    ms = jnp.max(am) + jnp.max(bm)
    ms = jnp.where(ms >= 0.0, ms, 0.2 * ms)
    mvec = jnp.full((L,), ms, jnp.float32)

    vv = vev[...]
    vebc = [jnp.full((L,), vv[f], jnp.float32) for f in range(DE)]

    iota = lax.iota(jnp.int32, L)

    def _chunk(c, _):
        ebase = wid * EPW + c * B
        pltpu.sync_copy(src_hbm.at[pl.ds(ebase, B)], srcv)
        pltpu.sync_copy(dst_hbm.at[pl.ds(ebase, B)], dstv)
        pltpu.sync_copy(ea_hbm.at[pl.ds(ebase, B)], eav)
        pltpu.async_copy(hext_hbm.at[srcv], rowsv, sem).wait()

        def _group(j, _):
            sv = srcv[pl.ds(j * L, L)]
            dv = dstv[pl.ds(j * L, L)]
            av = plsc.load_gather(asrcv, [sv])
            bv = plsc.load_gather(adstv, [dv])
            tv = iota + j * L
            aev = jnp.zeros((L,), jnp.float32)
            for f in range(DE):
                col = plsc.load_gather(eav, [tv, jnp.full((L,), f, jnp.int32)])
                aev = aev + col * vebc[f]
            al = av + bv + aev
            al = jnp.where(al >= 0.0, al, 0.2 * al)
            exbuf[pl.ds(j * L, L)] = jnp.exp(al - mvec)
            return 0

        lax.fori_loop(0, B // L, _group, 0)

        def _sgroup(j, _):
            exv = exbuf[pl.ds(j * L, L)]
            for i in range(L):
                t = j * L + i
                for cb in range(DX // L):
                    rowsv[t, pl.ds(cb * L, L)] = (rowsv[t, pl.ds(cb * L, L)]
                                                  * exv[i])
            return 0

        lax.fori_loop(0, B // L, _sgroup, 0)

        pltpu.sync_copy(rowsv, accsh.at[dstv], add=True)
        return 0

    lax.fori_loop(0, NCH, _chunk, 0)

    plsc.subcore_barrier()
    pltpu.sync_copy(accsh.at[pl.ds(sid * STRIPE, STRIPE)],
                    acc_hbm.at[cid, pl.ds(sid * STRIPE, STRIPE)])


_sc_edge = functools.partial(
    pl.kernel,
    _sc_edge_body,
    out_type=jax.ShapeDtypeStruct((NC, N, DX), jnp.float32),
    mesh=plsc.VectorSubcoreMesh(core_axis_name="c", subcore_axis_name="s",
                                num_cores=NC, num_subcores=NS),
    compiler_params=pltpu.CompilerParams(use_tc_tiling_on_sc=False,
                                         needs_layout_passes=False),
    scratch_types=[
        pltpu.VMEM((B,), jnp.int32),
        pltpu.VMEM((B,), jnp.int32),
        pltpu.VMEM((N,), jnp.float32),
        pltpu.VMEM((N,), jnp.float32),
        pltpu.VMEM((B, DE), jnp.float32),
        pltpu.VMEM((B, DX), jnp.float32),
        pltpu.VMEM((B,), jnp.float32),
        pltpu.VMEM((DE,), jnp.float32),
        pltpu.VMEM_SHARED((N, DX), jnp.float32),
        pltpu.SemaphoreType.DMA,
    ],
)()


def kernel(x, edge_index, edge_attr, batch,
           W1, att_src1, att_dst1, We1, att_edge1, b1,
           W2, att_src2, att_dst2, We2, att_edge2, b2,
           gamma, beta):
    src2d = edge_index[0]
    dst2d = edge_index[1]
    batch_row = batch.reshape(NB, 1, BN)
    ve1 = We1 @ att_edge1
    ve2 = We2 @ att_edge2

    hext1, asrc1, adst1 = _pre(x, W1, att_src1.reshape(D, 1),
                               att_dst1.reshape(D, 1))
    acc1 = _sc_edge(hext1, src2d, dst2d, edge_attr,
                    asrc1.reshape(N), adst1.reshape(N), ve1)
    x2, hext2, asrc2, adst2 = _mid(acc1, x, b1.reshape(1, D), W2,
                                   att_src2.reshape(D, 1),
                                   att_dst2.reshape(D, 1))
    acc2 = _sc_edge(hext2, src2d, dst2d, edge_attr,
                    asrc2.reshape(N), adst2.reshape(N), ve2)
    return _final(acc2, x2, b2.reshape(1, D), gamma.reshape(1, D),
                  beta.reshape(1, D), batch_row)


# speedup vs baseline: 19.3408x; 1.3331x over previous
import functools

import jax
import jax.numpy as jnp
from jax import lax
from jax.experimental import pallas as pl
from jax.experimental.pallas import tpu as pltpu
from jax.experimental.pallas import tpu_sc as plsc

N = 10000
E = 320000
D = 128
DE = 16
G = 16

DX = 144
NC, NS, L = 2, 16, 16
NW = NC * NS
EPW = E // NW
B = 80
NCH = EPW // B
STRIPE = N // NS
BN = 1000
NB = N // BN
BE = 2000
NEB = E // BE


def _silu(x):
    return x / (1.0 + jnp.exp(-x))


def _ext_cols(asrc_col):
    li = lax.broadcasted_iota(jnp.int32, (BN, DX - D), 1)
    return jnp.where(li == 0, 1.0, jnp.where(li == 1, asrc_col, 0.0))


def _ae_body(ea_ref, v1_ref, v2_ref, ae1_ref, ae2_ref, m1_ref, m2_ref):
    i = pl.program_id(0)
    ea = ea_ref[...]
    a1 = jnp.sum(ea * v1_ref[...], axis=1, keepdims=True)
    a2 = jnp.sum(ea * v2_ref[...], axis=1, keepdims=True)
    ae1_ref[...] = a1
    ae2_ref[...] = a2

    @pl.when(i == 0)
    def _():
        m1_ref[0, 0] = jnp.float32(-3.0e38)
        m2_ref[0, 0] = jnp.float32(-3.0e38)

    m1_ref[0, 0] = jnp.maximum(m1_ref[0, 0], jnp.max(a1))
    m2_ref[0, 0] = jnp.maximum(m2_ref[0, 0], jnp.max(a2))


def _ae(edge_attr, v1, v2):
    return pl.pallas_call(
        _ae_body,
        grid=(NEB,),
        in_specs=[
            pl.BlockSpec((BE, DE), lambda i: (i, 0)),
            pl.BlockSpec((1, DE), lambda i: (0, 0)),
            pl.BlockSpec((1, DE), lambda i: (0, 0)),
        ],
        out_specs=[
            pl.BlockSpec((BE, 1), lambda i: (i, 0)),
            pl.BlockSpec((BE, 1), lambda i: (i, 0)),
            pl.BlockSpec(memory_space=pltpu.SMEM),
            pl.BlockSpec(memory_space=pltpu.SMEM),
        ],
        out_shape=[
            jax.ShapeDtypeStruct((E, 1), jnp.float32),
            jax.ShapeDtypeStruct((E, 1), jnp.float32),
            jax.ShapeDtypeStruct((1, 1), jnp.float32),
            jax.ShapeDtypeStruct((1, 1), jnp.float32),
        ],
    )(edge_attr, v1, v2)


def _pre_body(x_ref, w_ref, as_ref, ad_ref, hext_ref, adst_ref,
              ms_ref, md_ref):
    i = pl.program_id(0)
    h = jnp.dot(x_ref[...], w_ref[...], preferred_element_type=jnp.float32)
    asrc = jnp.dot(h, as_ref[...], preferred_element_type=jnp.float32)
    adst = jnp.dot(h, ad_ref[...], preferred_element_type=jnp.float32)
    hext_ref[:, :D] = h
    hext_ref[:, D:] = _ext_cols(asrc)
    adst_ref[...] = adst

    @pl.when(i == 0)
    def _():
        ms_ref[0, 0] = jnp.float32(-3.0e38)
        md_ref[0, 0] = jnp.float32(-3.0e38)

    ms_ref[0, 0] = jnp.maximum(ms_ref[0, 0], jnp.max(asrc))
    md_ref[0, 0] = jnp.maximum(md_ref[0, 0], jnp.max(adst))


def _pre(x, W, a_src, a_dst):
    return pl.pallas_call(
        _pre_body,
        grid=(NB,),
        in_specs=[
            pl.BlockSpec((BN, D), lambda i: (i, 0)),
            pl.BlockSpec((D, D), lambda i: (0, 0)),
            pl.BlockSpec((D, 1), lambda i: (0, 0)),
            pl.BlockSpec((D, 1), lambda i: (0, 0)),
        ],
        out_specs=[
            pl.BlockSpec((BN, DX), lambda i: (i, 0)),
            pl.BlockSpec((BN, 1), lambda i: (i, 0)),
            pl.BlockSpec(memory_space=pltpu.SMEM),
            pl.BlockSpec(memory_space=pltpu.SMEM),
        ],
        out_shape=[
            jax.ShapeDtypeStruct((N, DX), jnp.float32),
            jax.ShapeDtypeStruct((N, 1), jnp.float32),
            jax.ShapeDtypeStruct((1, 1), jnp.float32),
            jax.ShapeDtypeStruct((1, 1), jnp.float32),
        ],
    )(x, W, a_src, a_dst)


def _mid_body(acc_ref, x_ref, b_ref, w_ref, as_ref, ad_ref,
              x2_ref, hext_ref, adst_ref, ms_ref, md_ref):
    i = pl.program_id(0)
    s = acc_ref[0] + acc_ref[1]
    o = s[:, :D] / (s[:, D:D + 1] + 1e-16) + b_ref[...]
    x2 = _silu(o) + x_ref[...]
    x2_ref[...] = x2
    h2 = jnp.dot(x2, w_ref[...], preferred_element_type=jnp.float32)
    asrc = jnp.dot(h2, as_ref[...], preferred_element_type=jnp.float32)
    adst = jnp.dot(h2, ad_ref[...], preferred_element_type=jnp.float32)
    hext_ref[:, :D] = h2
    hext_ref[:, D:] = _ext_cols(asrc)
    adst_ref[...] = adst

    @pl.when(i == 0)
    def _():
        ms_ref[0, 0] = jnp.float32(-3.0e38)
        md_ref[0, 0] = jnp.float32(-3.0e38)

    ms_ref[0, 0] = jnp.maximum(ms_ref[0, 0], jnp.max(asrc))
    md_ref[0, 0] = jnp.maximum(md_ref[0, 0], jnp.max(adst))


def _mid(acc, x, b, W, a_src, a_dst):
    return pl.pallas_call(
        _mid_body,
        grid=(NB,),
        in_specs=[
            pl.BlockSpec((2, BN, DX), lambda i: (0, i, 0)),
            pl.BlockSpec((BN, D), lambda i: (i, 0)),
            pl.BlockSpec((1, D), lambda i: (0, 0)),
            pl.BlockSpec((D, D), lambda i: (0, 0)),
            pl.BlockSpec((D, 1), lambda i: (0, 0)),
            pl.BlockSpec((D, 1), lambda i: (0, 0)),
        ],
        out_specs=[
            pl.BlockSpec((BN, D), lambda i: (i, 0)),
            pl.BlockSpec((BN, DX), lambda i: (i, 0)),
            pl.BlockSpec((BN, 1), lambda i: (i, 0)),
            pl.BlockSpec(memory_space=pltpu.SMEM),
            pl.BlockSpec(memory_space=pltpu.SMEM),
        ],
        out_shape=[
            jax.ShapeDtypeStruct((N, D), jnp.float32),
            jax.ShapeDtypeStruct((N, DX), jnp.float32),
            jax.ShapeDtypeStruct((N, 1), jnp.float32),
            jax.ShapeDtypeStruct((1, 1), jnp.float32),
            jax.ShapeDtypeStruct((1, 1), jnp.float32),
        ],
    )(acc, x, b, W, a_src, a_dst)


def _final_body(acc_ref, x_ref, b_ref, g_ref, be_ref, batch_ref,
                out_ref, pool_ref, cnt_ref):
    i = pl.program_id(0)
    s = acc_ref[0] + acc_ref[1]
    o = s[:, :D] / (s[:, D:D + 1] + 1e-16) + b_ref[...]
    h = _silu(o) + x_ref[...]
    mu = jnp.mean(h, axis=1, keepdims=True)
    var = jnp.mean((h - mu) ** 2, axis=1, keepdims=True)
    hn = g_ref[...] * (h - mu) * lax.rsqrt(var + 1e-5) + be_ref[...]
    onehot = (batch_ref[0] == lax.broadcasted_iota(jnp.int32, (G, BN), 0))
    onehot = onehot.astype(jnp.float32)
    part = jnp.dot(onehot, hn, preferred_element_type=jnp.float32)
    cpart = jnp.sum(onehot, axis=1)[:, None]

    @pl.when(i == 0)
    def _():
        pool_ref[...] = jnp.zeros((G, D), jnp.float32)
        cnt_ref[...] = jnp.zeros((G, 1), jnp.float32)

    pool_ref[...] += part
    cnt_ref[...] += cpart

    @pl.when(i == NB - 1)
    def _():
        out_ref[...] = pool_ref[...] / jnp.maximum(cnt_ref[...], 1.0)


def _final(acc, x2, b, gamma, beta, batch_row):
    return pl.pallas_call(
        _final_body,
        grid=(NB,),
        in_specs=[
            pl.BlockSpec((2, BN, DX), lambda i: (0, i, 0)),
            pl.BlockSpec((BN, D), lambda i: (i, 0)),
            pl.BlockSpec((1, D), lambda i: (0, 0)),
            pl.BlockSpec((1, D), lambda i: (0, 0)),
            pl.BlockSpec((1, D), lambda i: (0, 0)),
            pl.BlockSpec((1, 1, BN), lambda i: (i, 0, 0)),
        ],
        out_specs=pl.BlockSpec((G, D), lambda i: (0, 0)),
        out_shape=jax.ShapeDtypeStruct((G, D), jnp.float32),
        scratch_shapes=[
            pltpu.VMEM((G, D), jnp.float32),
            pltpu.VMEM((G, 1), jnp.float32),
        ],
    )(acc, x2, b, gamma, beta, batch_row)


def _sc_edge_body(hext_hbm, src_hbm, dst_hbm, ae_hbm, adst_hbm, m_hbm,
                  acc_hbm,
                  srcv0, srcv1, dstv0, dstv1, aev0, aev1, rows0, rows1,
                  adstv, mv, accsh, sg0, sg1, ss0, ss1):
    cid = lax.axis_index("c")
    sid = lax.axis_index("s")
    wid = sid * NC + cid
    eb = wid * EPW

    pltpu.sync_copy(adst_hbm, adstv)
    pltpu.sync_copy(m_hbm, mv)

    srcs = (srcv0, srcv1)
    dsts = (dstv0, dstv1)
    aes = (aev0, aev1)
    rows = (rows0, rows1)
    sgs = (sg0, sg1)
    sss = (ss0, ss1)

    zero16 = jnp.zeros((L,), jnp.float32)

    def _zrow(r, _):
        for cb in range(DX // L):
            rows0[r, pl.ds(cb * L, L)] = zero16
        return 0

    lax.fori_loop(0, B, _zrow, 0)
    for k in range(STRIPE // B):
        pltpu.sync_copy(rows0, accsh.at[pl.ds(sid * STRIPE + k * B, B)])
    rem = STRIPE % B
    if rem:
        pltpu.sync_copy(rows0.at[pl.ds(0, rem)],
                        accsh.at[pl.ds(sid * STRIPE + (STRIPE // B) * B, rem)])

    mvec = mv[...]
    iota = lax.iota(jnp.int32, L)
    col_as = jnp.full((L,), D + 1, jnp.int32)

    def _stage_start(c, p):
        base = eb + c * B
        pltpu.async_copy(src_hbm.at[pl.ds(base, B)], srcs[p], sss[p])
        pltpu.async_copy(dst_hbm.at[pl.ds(base, B)], dsts[p], sss[p])
        pltpu.async_copy(ae_hbm.at[pl.ds(base, B)], aes[p], sss[p])

    def _stage_wait(c, p):
        base = eb + c * B
        pltpu.make_async_copy(src_hbm.at[pl.ds(base, B)], srcs[p],
                              sss[p]).wait()
        pltpu.make_async_copy(dst_hbm.at[pl.ds(base, B)], dsts[p],
                              sss[p]).wait()
        pltpu.make_async_copy(ae_hbm.at[pl.ds(base, B)], aes[p],
                              sss[p]).wait()

    def _gather_start(p):
        pltpu.async_copy(hext_hbm.at[srcs[p]], rows[p], sgs[p])

    def _gather_wait(p):
        pltpu.make_async_copy(hext_hbm.at[srcs[p]], rows[p], sgs[p]).wait()

    def _compute_scatter(p):
        rv = rows[p]

        def _group(j, _):
            off = j * L
            dv = dsts[p][pl.ds(off, L)]
            bv = plsc.load_gather(adstv, [dv])
            av = plsc.load_gather(rv, [iota + off, col_as])
            al = av + bv + aes[p][pl.ds(off, L)]
            al = jnp.where(al >= 0.0, al, 0.2 * al)
            ex = jnp.exp(al - mvec)
            for i in range(L):
                t = off + i
                for cb in range(DX // L):
                    rv[t, pl.ds(cb * L, L)] = rv[t, pl.ds(cb * L, L)] * ex[i]
            return 0

        lax.fori_loop(0, B // L, _group, 0)
        pltpu.sync_copy(rv, accsh.at[dsts[p]], add=True)

    _stage_start(0, 0)
    _stage_wait(0, 0)
    _gather_start(0)
    _stage_start(1, 1)
    plsc.subcore_barrier()

    def _half(c, p):
        q = 1 - p
        _gather_wait(p)
        _stage_wait(c + 1, q)
        _gather_start(q)
        _compute_scatter(p)
        return c + 2

    def _iter(c2, _):
        c = c2 * 2
        _half(c, 0)
        _stage_start(c + 2, 0)
        _half(c + 1, 1)

        @pl.when(c + 3 < NCH)
        def _():
            _stage_start(c + 3, 1)

        return 0

    lax.fori_loop(0, (NCH - 1) // 2, _iter, 0)
    _gather_wait(0)
    _compute_scatter(0)

    plsc.subcore_barrier()
    pltpu.sync_copy(accsh.at[pl.ds(sid * STRIPE, STRIPE)],
                    acc_hbm.at[cid, pl.ds(sid * STRIPE, STRIPE)])


_sc_edge = pl.kernel(
    _sc_edge_body,
    out_type=jax.ShapeDtypeStruct((NC, N, DX), jnp.float32),
    mesh=plsc.VectorSubcoreMesh(core_axis_name="c", subcore_axis_name="s",
                                num_cores=NC, num_subcores=NS),
    compiler_params=pltpu.CompilerParams(use_tc_tiling_on_sc=False,
                                         needs_layout_passes=False),
    scratch_types=[
        pltpu.VMEM((B,), jnp.int32),
        pltpu.VMEM((B,), jnp.int32),
        pltpu.VMEM((B,), jnp.int32),
        pltpu.VMEM((B,), jnp.int32),
        pltpu.VMEM((B,), jnp.float32),
        pltpu.VMEM((B,), jnp.float32),
        pltpu.VMEM((B, DX), jnp.float32),
        pltpu.VMEM((B, DX), jnp.float32),
        pltpu.VMEM((N,), jnp.float32),
        pltpu.VMEM((L,), jnp.float32),
        pltpu.VMEM_SHARED((N, DX), jnp.float32),
        pltpu.SemaphoreType.DMA,
        pltpu.SemaphoreType.DMA,
        pltpu.SemaphoreType.DMA,
        pltpu.SemaphoreType.DMA,
    ],
)


def _lrelu(x):
    return jnp.where(x >= 0.0, x, 0.2 * x)


def kernel(x, edge_index, edge_attr, batch,
           W1, att_src1, att_dst1, We1, att_edge1, b1,
           W2, att_src2, att_dst2, We2, att_edge2, b2,
           gamma, beta):
    src = edge_index[0]
    dst = edge_index[1]
    batch_row = batch.reshape(NB, 1, BN)
    ve1 = (We1 @ att_edge1).reshape(1, DE)
    ve2 = (We2 @ att_edge2).reshape(1, DE)

    ae1, ae2, mae1, mae2 = _ae(edge_attr, ve1, ve2)
    ae1 = ae1.reshape(E)
    ae2 = ae2.reshape(E)

    hext1, adst1, ms1, md1 = _pre(x, W1, att_src1.reshape(D, 1),
                                  att_dst1.reshape(D, 1))
    m1 = _lrelu(ms1[0, 0] + md1[0, 0] + mae1[0, 0])
    acc1 = _sc_edge(hext1, src, dst, ae1, adst1.reshape(N),
                    jnp.full((L,), m1, jnp.float32))
    x2, hext2, adst2, ms2, md2 = _mid(acc1, x, b1.reshape(1, D), W2,
                                      att_src2.reshape(D, 1),
                                      att_dst2.reshape(D, 1))
    m2 = _lrelu(ms2[0, 0] + md2[0, 0] + mae2[0, 0])
    acc2 = _sc_edge(hext2, src, dst, ae2, adst2.reshape(N),
                    jnp.full((L,), m2, jnp.float32))
    return _final(acc2, x2, b2.reshape(1, D), gamma.reshape(1, D),
                  beta.reshape(1, D), batch_row)


# speedup vs baseline: 21.0040x; 1.0860x over previous
import functools

import jax
import jax.numpy as jnp
from jax import lax
from jax.experimental import pallas as pl
from jax.experimental.pallas import tpu as pltpu
from jax.experimental.pallas import tpu_sc as plsc

N = 10000
E = 320000
D = 128
DE = 16
G = 16

DX = 144
NC, NS, L = 2, 16, 16
NW = NC * NS
EPW = E // NW
B = 80
NCH = EPW // B
STRIPE = N // NS
BN = 1000
NB = N // BN
BE = 2000
NEB = E // BE


def _silu(x):
    return x / (1.0 + jnp.exp(-x))


def _ext_cols(asrc_col):
    li = lax.broadcasted_iota(jnp.int32, (BN, DX - D), 1)
    return jnp.where(li == 0, 1.0, jnp.where(li == 1, asrc_col, 0.0))


def _ae_body(ea_ref, v1_ref, v2_ref, ae1_ref, ae2_ref, m1_ref, m2_ref):
    i = pl.program_id(0)
    ea = ea_ref[...]
    a1 = jnp.sum(ea * v1_ref[...], axis=1, keepdims=True)
    a2 = jnp.sum(ea * v2_ref[...], axis=1, keepdims=True)
    ae1_ref[...] = a1
    ae2_ref[...] = a2

    @pl.when(i == 0)
    def _():
        m1_ref[0, 0] = jnp.float32(-3.0e38)
        m2_ref[0, 0] = jnp.float32(-3.0e38)

    m1_ref[0, 0] = jnp.maximum(m1_ref[0, 0], jnp.max(a1))
    m2_ref[0, 0] = jnp.maximum(m2_ref[0, 0], jnp.max(a2))


def _ae(edge_attr, v1, v2):
    return pl.pallas_call(
        _ae_body,
        grid=(NEB,),
        in_specs=[
            pl.BlockSpec((BE, DE), lambda i: (i, 0)),
            pl.BlockSpec((1, DE), lambda i: (0, 0)),
            pl.BlockSpec((1, DE), lambda i: (0, 0)),
        ],
        out_specs=[
            pl.BlockSpec((BE, 1), lambda i: (i, 0)),
            pl.BlockSpec((BE, 1), lambda i: (i, 0)),
            pl.BlockSpec(memory_space=pltpu.SMEM),
            pl.BlockSpec(memory_space=pltpu.SMEM),
        ],
        out_shape=[
            jax.ShapeDtypeStruct((E, 1), jnp.float32),
            jax.ShapeDtypeStruct((E, 1), jnp.float32),
            jax.ShapeDtypeStruct((1, 1), jnp.float32),
            jax.ShapeDtypeStruct((1, 1), jnp.float32),
        ],
    )(edge_attr, v1, v2)


def _pre_body(x_ref, w_ref, as_ref, ad_ref, hext_ref, adst_ref,
              ms_ref, md_ref):
    i = pl.program_id(0)
    h = jnp.dot(x_ref[...], w_ref[...], preferred_element_type=jnp.float32)
    asrc = jnp.dot(h, as_ref[...], preferred_element_type=jnp.float32)
    adst = jnp.dot(h, ad_ref[...], preferred_element_type=jnp.float32)
    hext_ref[:, :D] = h
    hext_ref[:, D:] = _ext_cols(asrc)
    adst_ref[...] = adst

    @pl.when(i == 0)
    def _():
        ms_ref[0, 0] = jnp.float32(-3.0e38)
        md_ref[0, 0] = jnp.float32(-3.0e38)

    ms_ref[0, 0] = jnp.maximum(ms_ref[0, 0], jnp.max(asrc))
    md_ref[0, 0] = jnp.maximum(md_ref[0, 0], jnp.max(adst))


def _pre(x, W, a_src, a_dst):
    return pl.pallas_call(
        _pre_body,
        grid=(NB,),
        in_specs=[
            pl.BlockSpec((BN, D), lambda i: (i, 0)),
            pl.BlockSpec((D, D), lambda i: (0, 0)),
            pl.BlockSpec((D, 1), lambda i: (0, 0)),
            pl.BlockSpec((D, 1), lambda i: (0, 0)),
        ],
        out_specs=[
            pl.BlockSpec((BN, DX), lambda i: (i, 0)),
            pl.BlockSpec((BN, 1), lambda i: (i, 0)),
            pl.BlockSpec(memory_space=pltpu.SMEM),
            pl.BlockSpec(memory_space=pltpu.SMEM),
        ],
        out_shape=[
            jax.ShapeDtypeStruct((N, DX), jnp.float32),
            jax.ShapeDtypeStruct((N, 1), jnp.float32),
            jax.ShapeDtypeStruct((1, 1), jnp.float32),
            jax.ShapeDtypeStruct((1, 1), jnp.float32),
        ],
    )(x, W, a_src, a_dst)


def _mid_body(acc_ref, x_ref, b_ref, w_ref, as_ref, ad_ref,
              x2_ref, hext_ref, adst_ref, ms_ref, md_ref):
    i = pl.program_id(0)
    s = acc_ref[0] + acc_ref[1]
    o = s[:, :D] / (s[:, D:D + 1] + 1e-16) + b_ref[...]
    x2 = _silu(o) + x_ref[...]
    x2_ref[...] = x2
    h2 = jnp.dot(x2, w_ref[...], preferred_element_type=jnp.float32)
    asrc = jnp.dot(h2, as_ref[...], preferred_element_type=jnp.float32)
    adst = jnp.dot(h2, ad_ref[...], preferred_element_type=jnp.float32)
    hext_ref[:, :D] = h2
    hext_ref[:, D:] = _ext_cols(asrc)
    adst_ref[...] = adst

    @pl.when(i == 0)
    def _():
        ms_ref[0, 0] = jnp.float32(-3.0e38)
        md_ref[0, 0] = jnp.float32(-3.0e38)

    ms_ref[0, 0] = jnp.maximum(ms_ref[0, 0], jnp.max(asrc))
    md_ref[0, 0] = jnp.maximum(md_ref[0, 0], jnp.max(adst))


def _mid(acc, x, b, W, a_src, a_dst):
    return pl.pallas_call(
        _mid_body,
        grid=(NB,),
        in_specs=[
            pl.BlockSpec((2, BN, DX), lambda i: (0, i, 0)),
            pl.BlockSpec((BN, D), lambda i: (i, 0)),
            pl.BlockSpec((1, D), lambda i: (0, 0)),
            pl.BlockSpec((D, D), lambda i: (0, 0)),
            pl.BlockSpec((D, 1), lambda i: (0, 0)),
            pl.BlockSpec((D, 1), lambda i: (0, 0)),
        ],
        out_specs=[
            pl.BlockSpec((BN, D), lambda i: (i, 0)),
            pl.BlockSpec((BN, DX), lambda i: (i, 0)),
            pl.BlockSpec((BN, 1), lambda i: (i, 0)),
            pl.BlockSpec(memory_space=pltpu.SMEM),
            pl.BlockSpec(memory_space=pltpu.SMEM),
        ],
        out_shape=[
            jax.ShapeDtypeStruct((N, D), jnp.float32),
            jax.ShapeDtypeStruct((N, DX), jnp.float32),
            jax.ShapeDtypeStruct((N, 1), jnp.float32),
            jax.ShapeDtypeStruct((1, 1), jnp.float32),
            jax.ShapeDtypeStruct((1, 1), jnp.float32),
        ],
    )(acc, x, b, W, a_src, a_dst)


def _final_body(acc_ref, x_ref, b_ref, g_ref, be_ref, batch_ref,
                out_ref, pool_ref, cnt_ref):
    i = pl.program_id(0)
    s = acc_ref[0] + acc_ref[1]
    o = s[:, :D] / (s[:, D:D + 1] + 1e-16) + b_ref[...]
    h = _silu(o) + x_ref[...]
    mu = jnp.mean(h, axis=1, keepdims=True)
    var = jnp.mean((h - mu) ** 2, axis=1, keepdims=True)
    hn = g_ref[...] * (h - mu) * lax.rsqrt(var + 1e-5) + be_ref[...]
    onehot = (batch_ref[0] == lax.broadcasted_iota(jnp.int32, (G, BN), 0))
    onehot = onehot.astype(jnp.float32)
    part = jnp.dot(onehot, hn, preferred_element_type=jnp.float32)
    cpart = jnp.sum(onehot, axis=1)[:, None]

    @pl.when(i == 0)
    def _():
        pool_ref[...] = jnp.zeros((G, D), jnp.float32)
        cnt_ref[...] = jnp.zeros((G, 1), jnp.float32)

    pool_ref[...] += part
    cnt_ref[...] += cpart

    @pl.when(i == NB - 1)
    def _():
        out_ref[...] = pool_ref[...] / jnp.maximum(cnt_ref[...], 1.0)


def _final(acc, x2, b, gamma, beta, batch_row):
    return pl.pallas_call(
        _final_body,
        grid=(NB,),
        in_specs=[
            pl.BlockSpec((2, BN, DX), lambda i: (0, i, 0)),
            pl.BlockSpec((BN, D), lambda i: (i, 0)),
            pl.BlockSpec((1, D), lambda i: (0, 0)),
            pl.BlockSpec((1, D), lambda i: (0, 0)),
            pl.BlockSpec((1, D), lambda i: (0, 0)),
            pl.BlockSpec((1, 1, BN), lambda i: (i, 0, 0)),
        ],
        out_specs=pl.BlockSpec((G, D), lambda i: (0, 0)),
        out_shape=jax.ShapeDtypeStruct((G, D), jnp.float32),
        scratch_shapes=[
            pltpu.VMEM((G, D), jnp.float32),
            pltpu.VMEM((G, 1), jnp.float32),
        ],
    )(acc, x2, b, gamma, beta, batch_row)


def _sc_edge_body(hext_hbm, src_hbm, dst_hbm, ae_hbm, adst_hbm, m_hbm,
                  acc_hbm,
                  srcv0, srcv1, dstv0, dstv1, aev0, aev1, rows0, rows1,
                  dsc0, dsc1, adstv, mv, accsh, sg0, sg1, ss0, ss1,
                  sc0, sc1):
    cid = lax.axis_index("c")
    sid = lax.axis_index("s")
    wid = sid * NC + cid
    eb = wid * EPW

    pltpu.sync_copy(adst_hbm, adstv)
    pltpu.sync_copy(m_hbm, mv)

    srcs = (srcv0, srcv1)
    dsts = (dstv0, dstv1)
    aes = (aev0, aev1)
    rows = (rows0, rows1)
    dscs = (dsc0, dsc1)
    sgs = (sg0, sg1)
    sss = (ss0, ss1)
    scs = (sc0, sc1)

    zero16 = jnp.zeros((L,), jnp.float32)

    def _zrow(r, _):
        for cb in range(DX // L):
            rows0[r, pl.ds(cb * L, L)] = zero16
        return 0

    lax.fori_loop(0, B, _zrow, 0)
    for k in range(STRIPE // B):
        pltpu.sync_copy(rows0, accsh.at[pl.ds(sid * STRIPE + k * B, B)])
    rem = STRIPE % B
    if rem:
        pltpu.sync_copy(rows0.at[pl.ds(0, rem)],
                        accsh.at[pl.ds(sid * STRIPE + (STRIPE // B) * B, rem)])

    mvec = mv[...]
    iota = lax.iota(jnp.int32, L)
    col_as = jnp.full((L,), D + 1, jnp.int32)

    def _stage_start(c, p):
        base = eb + c * B
        pltpu.async_copy(src_hbm.at[pl.ds(base, B)], srcs[p], sss[p])
        pltpu.async_copy(dst_hbm.at[pl.ds(base, B)], dsts[p], sss[p])
        pltpu.async_copy(ae_hbm.at[pl.ds(base, B)], aes[p], sss[p])

    def _stage_wait(c, p):
        base = eb + c * B
        pltpu.make_async_copy(src_hbm.at[pl.ds(base, B)], srcs[p],
                              sss[p]).wait()
        pltpu.make_async_copy(dst_hbm.at[pl.ds(base, B)], dsts[p],
                              sss[p]).wait()
        pltpu.make_async_copy(ae_hbm.at[pl.ds(base, B)], aes[p],
                              sss[p]).wait()

    def _gather_start(p):
        pltpu.async_copy(hext_hbm.at[srcs[p]], rows[p], sgs[p])

    def _gather_wait(p):
        pltpu.make_async_copy(hext_hbm.at[srcs[p]], rows[p], sgs[p]).wait()

    def _compute_scatter(p):
        rv = rows[p]

        def _group(j, _):
            off = j * L
            dv = dsts[p][pl.ds(off, L)]
            bv = plsc.load_gather(adstv, [dv])
            av = plsc.load_gather(rv, [iota + off, col_as])
            al = av + bv + aes[p][pl.ds(off, L)]
            al = jnp.where(al >= 0.0, al, 0.2 * al)
            ex = jnp.exp(al - mvec)
            for i in range(L):
                t = off + i
                for cb in range(DX // L):
                    rv[t, pl.ds(cb * L, L)] = rv[t, pl.ds(cb * L, L)] * ex[i]
            return 0

        lax.fori_loop(0, B // L, _group, 0)
        for k in range(B // L):
            dscs[p][pl.ds(k * L, L)] = dsts[p][pl.ds(k * L, L)]
        pltpu.async_copy(rv, accsh.at[dscs[p]], scs[p], add=True)

    def _scatter_wait(p):
        pltpu.make_async_copy(rows[p], accsh.at[dscs[p]], scs[p]).wait()

    _stage_start(0, 0)
    _stage_wait(0, 0)
    _gather_start(0)
    _stage_start(1, 1)
    plsc.subcore_barrier()

    def _half(c, p, wait_prev_scatter):
        q = 1 - p
        _gather_wait(p)
        _stage_wait(c + 1, q)
        if wait_prev_scatter:
            _scatter_wait(q)
        _gather_start(q)
        _compute_scatter(p)

    def _iter(c2, _):
        c = c2 * 2
        _half(c, 0, True)
        _stage_start(c + 2, 0)
        _half(c + 1, 1, True)

        @pl.when(c + 3 < NCH)
        def _():
            _stage_start(c + 3, 1)

        return 0

    _half(0, 0, False)
    _stage_start(2, 0)
    _half(1, 1, True)
    _stage_start(3, 1)
    def _iter_shift(c2, _):
        return _iter(c2 + 1, _)

    lax.fori_loop(0, (NCH - 1) // 2 - 1, _iter_shift, 0)
    _gather_wait(0)
    _scatter_wait(1)
    _compute_scatter(0)
    _scatter_wait(0)

    plsc.subcore_barrier()
    pltpu.sync_copy(accsh.at[pl.ds(sid * STRIPE, STRIPE)],
                    acc_hbm.at[cid, pl.ds(sid * STRIPE, STRIPE)])


_sc_edge = pl.kernel(
    _sc_edge_body,
    out_type=jax.ShapeDtypeStruct((NC, N, DX), jnp.float32),
    mesh=plsc.VectorSubcoreMesh(core_axis_name="c", subcore_axis_name="s",
                                num_cores=NC, num_subcores=NS),
    compiler_params=pltpu.CompilerParams(use_tc_tiling_on_sc=False,
                                         needs_layout_passes=False),
    scratch_types=[
        pltpu.VMEM((B,), jnp.int32),
        pltpu.VMEM((B,), jnp.int32),
        pltpu.VMEM((B,), jnp.int32),
        pltpu.VMEM((B,), jnp.int32),
        pltpu.VMEM((B,), jnp.float32),
        pltpu.VMEM((B,), jnp.float32),
        pltpu.VMEM((B, DX), jnp.float32),
        pltpu.VMEM((B, DX), jnp.float32),
        pltpu.VMEM((B,), jnp.int32),
        pltpu.VMEM((B,), jnp.int32),
        pltpu.VMEM((N,), jnp.float32),
        pltpu.VMEM((L,), jnp.float32),
        pltpu.VMEM_SHARED((N, DX), jnp.float32),
        pltpu.SemaphoreType.DMA,
        pltpu.SemaphoreType.DMA,
        pltpu.SemaphoreType.DMA,
        pltpu.SemaphoreType.DMA,
        pltpu.SemaphoreType.DMA,
        pltpu.SemaphoreType.DMA,
    ],
)


def _lrelu(x):
    return jnp.where(x >= 0.0, x, 0.2 * x)


def kernel(x, edge_index, edge_attr, batch,
           W1, att_src1, att_dst1, We1, att_edge1, b1,
           W2, att_src2, att_dst2, We2, att_edge2, b2,
           gamma, beta):
    src = edge_index[0]
    dst = edge_index[1]
    batch_row = batch.reshape(NB, 1, BN)
    ve1 = (We1 @ att_edge1).reshape(1, DE)
    ve2 = (We2 @ att_edge2).reshape(1, DE)

    ae1, ae2, mae1, mae2 = _ae(edge_attr, ve1, ve2)
    ae1 = ae1.reshape(E)
    ae2 = ae2.reshape(E)

    hext1, adst1, ms1, md1 = _pre(x, W1, att_src1.reshape(D, 1),
                                  att_dst1.reshape(D, 1))
    m1 = _lrelu(ms1[0, 0] + md1[0, 0] + mae1[0, 0])
    acc1 = _sc_edge(hext1, src, dst, ae1, adst1.reshape(N),
                    jnp.full((L,), m1, jnp.float32))
    x2, hext2, adst2, ms2, md2 = _mid(acc1, x, b1.reshape(1, D), W2,
                                      att_src2.reshape(D, 1),
                                      att_dst2.reshape(D, 1))
    m2 = _lrelu(ms2[0, 0] + md2[0, 0] + mae2[0, 0])
    acc2 = _sc_edge(hext2, src, dst, ae2, adst2.reshape(N),
                    jnp.full((L,), m2, jnp.float32))
    return _final(acc2, x2, b2.reshape(1, D), gamma.reshape(1, D),
                  beta.reshape(1, D), batch_row)
